# Initial kernel scaffold; baseline (speedup 1.0000x reference)
import jax
import jax.numpy as jnp
from jax.experimental import pallas as pl


def kernel(tokens, table, pos_emb):
    raise NotImplementedError("write your pallas kernel here")


# speedup vs baseline: 2.5353x; 2.5353x over previous
import functools

import jax
import jax.numpy as jnp
from jax import lax
from jax.experimental import pallas as pl
from jax.experimental.pallas import tpu as pltpu
from jax.experimental.pallas import tpu_sc as plsc

VOCAB = 100000
D = 64
B = 4096
L = 200
N = B * L

NC = 2
NS = 16
NW = NC * NS
R = N // NW
CH = 128
K = 8
STEP = CH * K
NSTEPS = R // STEP


def _sc_kernel(tok_hbm, table_hbm, pos_hbm, out_hbm, idx_v, rows_v, pos_v, sem):
    wid = lax.axis_index("s") * NC + lax.axis_index("c")
    base = wid * R

    pltpu.sync_copy(pos_hbm, pos_v)

    def step_body(s, carry):
        off = base + s * STEP
        pltpu.sync_copy(tok_hbm.at[pl.ds(off, STEP)], idx_v)
        copies = [
            pltpu.async_copy(
                table_hbm.at[idx_v.at[pl.ds(j * CH, CH)]],
                rows_v.at[pl.ds(j * CH, CH)],
                sem,
            )
            for j in range(K)
        ]
        for c in copies:
            c.wait()

        def row_body(i, c2):
            p = lax.rem(off + i, L)
            for d in range(D // 16):
                sl = pl.ds(d * 16, 16)
                e = rows_v[i, sl]
                rows_v[i, sl] = e + e + pos_v[p, sl]
            return c2

        lax.fori_loop(0, STEP, row_body, 0, unroll=False)
        pltpu.sync_copy(rows_v, out_hbm.at[pl.ds(off, STEP)])
        return carry

    lax.fori_loop(0, NSTEPS, step_body, 0, unroll=False)


def kernel(tokens, table, pos_emb):
    tok_flat = tokens.reshape(N).astype(jnp.int32)
    mesh = plsc.VectorSubcoreMesh(core_axis_name="c", subcore_axis_name="s")
    run = functools.partial(
        pl.kernel,
        mesh=mesh,
        out_type=jax.ShapeDtypeStruct((N, D), jnp.float32),
        scratch_types=[
            pltpu.VMEM((STEP,), jnp.int32),
            pltpu.VMEM((STEP, D), jnp.float32),
            pltpu.VMEM((L, D), jnp.float32),
            pltpu.SemaphoreType.DMA,
        ],
        compiler_params=pltpu.CompilerParams(use_tc_tiling_on_sc=False),
    )(_sc_kernel)
    out = run(tok_flat, table, pos_emb)
    return out.reshape(B, L, D)


# speedup vs baseline: 3.6690x; 1.4471x over previous
import functools

import jax
import jax.numpy as jnp
from jax import lax
from jax.experimental import pallas as pl
from jax.experimental.pallas import tpu as pltpu
from jax.experimental.pallas import tpu_sc as plsc

VOCAB = 100000
D = 64
B = 4096
L = 200
N = B * L

NC = 2
NS = 16
NW = NC * NS
R = N // NW
CH = 100
K = 8
STEP = CH * K
NSTEPS = R // STEP
REPS = STEP // L


def _fill_pos(buf, pos_sh):
    pltpu.sync_copy(pos_sh, buf)


def _gather2(table_hbm, idx, buf, sem):
    copies = []
    for j in range(K):
        src = table_hbm.at[idx.at[j]]
        dst = buf.at[pl.ds(j * CH, CH)]
        copies.append(pltpu.async_copy(src, dst, sem, add=True))
        copies.append(pltpu.async_copy(src, dst, sem, add=True))
    return copies


def _sc_kernel(tok_hbm, table_hbm, pos_hbm, out_hbm,
               idx0, idx1, buf0, buf1, pos_sh, sem_g, sem_s):
    sid = lax.axis_index("s")
    wid = sid * NC + lax.axis_index("c")
    base = wid * R
    ibase = wid * (R // CH)

    def wait_scatter(buf):
        pltpu.make_async_copy(buf, out_hbm.at[pl.ds(base, STEP)], sem_s).wait()

    def stage(idx, buf, s):
        _fill_pos(buf, pos_sh)
        pltpu.sync_copy(tok_hbm.at[pl.ds(ibase + s * K, K)], idx)

    def scatter(buf, s):
        return pltpu.async_copy(buf, out_hbm.at[pl.ds(base + s * STEP, STEP)],
                                sem_s)

    @pl.when(sid == 0)
    def _():
        for j in range(REPS):
            pltpu.sync_copy(pos_hbm, pos_sh.at[pl.ds(j * L, L)])

    plsc.subcore_barrier()

    stage(idx0, buf0, 0)
    g = _gather2(table_hbm, idx0, buf0, sem_g)
    stage(idx1, buf1, 1)
    for c in g:
        c.wait()
    scatter(buf0, 0)
    g = _gather2(table_hbm, idx1, buf1, sem_g)
    for c in g:
        c.wait()
    scatter(buf1, 1)

    def body(i, carry):
        s0 = 2 * i
        s1 = s0 + 1
        wait_scatter(buf0)
        stage(idx0, buf0, s0)
        g0 = _gather2(table_hbm, idx0, buf0, sem_g)
        wait_scatter(buf1)
        stage(idx1, buf1, s1)
        for c in g0:
            c.wait()
        scatter(buf0, s0)
        g1 = _gather2(table_hbm, idx1, buf1, sem_g)
        for c in g1:
            c.wait()
        scatter(buf1, s1)
        return carry

    lax.fori_loop(1, NSTEPS // 2, body, 0, unroll=False)

    wait_scatter(buf0)
    wait_scatter(buf1)


def kernel(tokens, table, pos_emb):
    tok2 = tokens.reshape(N // CH, CH).astype(jnp.int32)
    mesh = plsc.VectorSubcoreMesh(core_axis_name="c", subcore_axis_name="s")
    run = functools.partial(
        pl.kernel,
        mesh=mesh,
        out_type=jax.ShapeDtypeStruct((N, D), jnp.float32),
        scratch_types=[
            pltpu.VMEM((K, CH), jnp.int32),
            pltpu.VMEM((K, CH), jnp.int32),
            pltpu.VMEM((STEP, D), jnp.float32),
            pltpu.VMEM((STEP, D), jnp.float32),
            pltpu.VMEM_SHARED((STEP, D), jnp.float32),
            pltpu.SemaphoreType.DMA,
            pltpu.SemaphoreType.DMA,
        ],
        compiler_params=pltpu.CompilerParams(use_tc_tiling_on_sc=False),
    )(_sc_kernel)
    out = run(tok2, table, pos_emb)
    return out.reshape(B, L, D)
